# baseline (speedup 1.0000x reference)
import functools

import jax
import jax.numpy as jnp
from jax import lax
from jax.experimental import pallas as pl
from jax.experimental.pallas import tpu as pltpu
from jax.experimental.pallas import tpu_sc as plsc

N_ROWS = 8192
N_CH = 2048
NC = 2
NS = 16
L = 16
NW = NC * NS
ROWS_PER_W = N_ROWS // NW
RB = 4
NBLK = ROWS_PER_W // RB
NBUF = 4
NG = NBLK // NBUF
NJ = N_CH // L

_mesh = plsc.VectorSubcoreMesh(
    core_axis_name="c", subcore_axis_name="s", num_cores=NC, num_subcores=NS
)


@functools.partial(
    pl.kernel,
    mesh=_mesh,
    compiler_params=pltpu.CompilerParams(needs_layout_passes=False),
    out_type=jax.ShapeDtypeStruct((N_ROWS, N_CH), jnp.float32),
    scratch_types=(
        [pltpu.VMEM((N_CH,), jnp.int32)]
        + [pltpu.VMEM((RB, N_CH), jnp.float32) for _ in range(2 * NBUF)]
        + [pltpu.SemaphoreType.DMA for _ in range(2 * NBUF)]
    ),
)
def _permute(x_hbm, perm_hbm, out_hbm, perm_v, *bufs):
    wid = lax.axis_index("s") * NC + lax.axis_index("c")
    base = wid * ROWS_PER_W
    pltpu.sync_copy(perm_hbm, perm_v)

    ins = bufs[:NBUF]
    outs = bufs[NBUF:2 * NBUF]
    sins = bufs[2 * NBUF:3 * NBUF]
    souts = bufs[3 * NBUF:]

    def start_in(blk, b):
        src = x_hbm.at[pl.ds(base + blk * RB, RB)]
        pltpu.make_async_copy(src, ins[b], sins[b]).start()

    def wait_in(b):
        src = x_hbm.at[pl.ds(base, RB)]
        pltpu.make_async_copy(src, ins[b], sins[b]).wait()

    def start_out(blk, b):
        dst = out_hbm.at[pl.ds(base + blk * RB, RB)]
        pltpu.make_async_copy(outs[b], dst, souts[b]).start()

    def wait_out(b):
        dst = out_hbm.at[pl.ds(base, RB)]
        pltpu.make_async_copy(outs[b], dst, souts[b]).wait()

    def compute(b):
        in_v = ins[b]
        out_v = outs[b]

        @plsc.parallel_loop(0, NJ, 1, unroll=2)
        def jloop(jc):
            idx = perm_v[pl.ds(jc * L, L)]
            vals = [
                plsc.load_gather(
                    in_v, [jnp.full((L,), r, jnp.int32), idx])
                for r in range(RB)
            ]
            for r in range(RB):
                out_v[r, pl.ds(jc * L, L)] = vals[r]

    for b in range(NBUF):
        start_in(b, b)

    for b in range(NBUF):
        wait_in(b)
        compute(b)
        start_out(b, b)
        start_in(NBUF + b, b)

    def steady(g, carry):
        for b in range(NBUF):
            blk = g * NBUF + b
            wait_in(b)
            wait_out(b)
            compute(b)
            start_out(blk, b)
            start_in(blk + NBUF, b)
        return carry

    lax.fori_loop(1, NG - 1, steady, 0)

    for b in range(NBUF):
        blk = (NG - 1) * NBUF + b
        wait_in(b)
        wait_out(b)
        compute(b)
        start_out(blk, b)

    for b in range(NBUF):
        wait_out(b)


def kernel(x, perm):
    return _permute(x, perm.astype(jnp.int32))


# speedup vs baseline: 1.0047x; 1.0047x over previous
import functools

import jax
import jax.numpy as jnp
from jax import lax
from jax.experimental import pallas as pl
from jax.experimental.pallas import tpu as pltpu
from jax.experimental.pallas import tpu_sc as plsc

N_ROWS = 8192
N_CH = 2048
NC = 2
NS = 16
L = 16
NW = NC * NS
ROWS_PER_W = N_ROWS // NW
RB = 4
NBLK = ROWS_PER_W // RB
NBUF = 4
NG = NBLK // NBUF
NJ = N_CH // L

_mesh = plsc.VectorSubcoreMesh(
    core_axis_name="c", subcore_axis_name="s", num_cores=NC, num_subcores=NS
)


@functools.partial(
    pl.kernel,
    mesh=_mesh,
    compiler_params=pltpu.CompilerParams(needs_layout_passes=False),
    out_type=jax.ShapeDtypeStruct((N_ROWS, N_CH), jnp.float32),
    scratch_types=(
        [pltpu.VMEM((N_CH,), jnp.int32)]
        + [pltpu.VMEM((RB, N_CH), jnp.float32) for _ in range(2 * NBUF)]
        + [pltpu.SemaphoreType.DMA for _ in range(2 * NBUF)]
    ),
)
def _permute(x_hbm, perm_hbm, out_hbm, perm_v, *bufs):
    wid = lax.axis_index("s") * NC + lax.axis_index("c")
    base = wid * ROWS_PER_W

    ins = bufs[:NBUF]
    outs = bufs[NBUF:2 * NBUF]
    sins = bufs[2 * NBUF:3 * NBUF]
    souts = bufs[3 * NBUF:]

    def start_in(blk, b):
        src = x_hbm.at[pl.ds(base + blk * RB, RB)]
        pltpu.make_async_copy(src, ins[b], sins[b]).start()

    def wait_in(b):
        src = x_hbm.at[pl.ds(base, RB)]
        pltpu.make_async_copy(src, ins[b], sins[b]).wait()

    def start_out(blk, b):
        dst = out_hbm.at[pl.ds(base + blk * RB, RB)]
        pltpu.make_async_copy(outs[b], dst, souts[b]).start()

    def wait_out(b):
        dst = out_hbm.at[pl.ds(base, RB)]
        pltpu.make_async_copy(outs[b], dst, souts[b]).wait()

    def compute(b):
        in_v = ins[b]
        out_v = outs[b]

        @plsc.parallel_loop(0, NJ, 1, unroll=2)
        def jloop(jc):
            idx = perm_v[pl.ds(jc * L, L)]
            vals = [
                plsc.load_gather(
                    in_v, [jnp.full((L,), r, jnp.int32), idx])
                for r in range(RB)
            ]
            for r in range(RB):
                out_v[r, pl.ds(jc * L, L)] = vals[r]

    for b in range(NBUF):
        start_in(b, b)
    pltpu.sync_copy(perm_hbm, perm_v)

    for b in range(NBUF):
        wait_in(b)
        compute(b)
        start_out(b, b)
        start_in(NBUF + b, b)

    def steady(g, carry):
        for b in range(NBUF):
            blk = g * NBUF + b
            wait_in(b)
            wait_out(b)
            compute(b)
            start_out(blk, b)
            start_in(blk + NBUF, b)
        return carry

    lax.fori_loop(1, NG - 1, steady, 0)

    for b in range(NBUF):
        blk = (NG - 1) * NBUF + b
        wait_in(b)
        wait_out(b)
        compute(b)
        start_out(blk, b)

    for b in range(NBUF):
        wait_out(b)


def kernel(x, perm):
    return _permute(x, perm.astype(jnp.int32))
